# baseline (speedup 1.0000x reference)
import functools

import jax
import jax.numpy as jnp
from jax import lax
from jax.experimental import pallas as pl
from jax.experimental.pallas import tpu as pltpu
from jax.experimental.pallas import tpu_sc as plsc

NC = 2
NS = 16
NW = NC * NS
CH = 128
LANES = 16

N = 10000
D = 128
DEG_PAD = 10240
DEG_SLICE = DEG_PAD // NS
ACC_ROWS = 10240
RPT = ACC_ROWS // NS
ZR = 128


def _sc_mesh():
    return plsc.VectorSubcoreMesh(core_axis_name="c", subcore_axis_name="s")


def _zero_1d(ref, n):
    def body(i, _):
        ref[pl.ds(i * LANES, LANES)] = jnp.zeros((LANES,), jnp.float32)
        return 0
    lax.fori_loop(0, n // LANES, body, 0)


def _deg_body(nch, colp_hbm, out_hbm, colv, ones_v, zb, deg_sh):
    cid = lax.axis_index("c")
    sid = lax.axis_index("s")
    wid = sid * NC + cid
    pltpu.sync_copy(colp_hbm.at[wid], colv)
    _zero_1d(zb, DEG_SLICE)
    pltpu.sync_copy(zb, deg_sh.at[pl.ds(sid * DEG_SLICE, DEG_SLICE)])
    def ones_body(i, _):
        ones_v[pl.ds(i * LANES, LANES)] = jnp.ones((LANES,), jnp.float32)
        return 0
    lax.fori_loop(0, CH // LANES, ones_body, 0)
    plsc.subcore_barrier()
    def ch_body(c, _):
        pltpu.sync_copy(ones_v, deg_sh.at[colv.at[c]], add=True)
        return 0
    lax.fori_loop(0, nch, ch_body, 0)
    plsc.subcore_barrier()
    pltpu.sync_copy(deg_sh.at[pl.ds(sid * DEG_SLICE, DEG_SLICE)], zb)
    pltpu.sync_copy(zb, out_hbm.at[cid, pl.ds(sid * DEG_SLICE, DEG_SLICE)])


def _sc_deg(colp):
    nch = colp.shape[1]
    fn = pl.kernel(
        functools.partial(_deg_body, nch),
        out_type=jax.ShapeDtypeStruct((NC, DEG_PAD), jnp.float32),
        mesh=_sc_mesh(),
        scratch_types=[
            pltpu.VMEM((nch, CH), jnp.int32),
            pltpu.VMEM((CH,), jnp.float32),
            pltpu.VMEM((DEG_SLICE,), jnp.float32),
            pltpu.VMEM_SHARED((DEG_PAD,), jnp.float32),
        ],
    )
    return fn(colp)


GRP = 16


def _side_body(nch, xs_hbm, rowp_hbm, colp_hbm, wp_hbm, out_hbm,
               rowv, colv, wv, rows, acc, gsem, ssem):
    cid = lax.axis_index("c")
    sid = lax.axis_index("s")
    wid = sid * NC + cid
    ng = nch // GRP

    def stage(g, buf):
        sl = pl.ds(pl.multiple_of(g * GRP, GRP), GRP)
        pltpu.sync_copy(rowp_hbm.at[wid, sl], rowv.at[buf])
        pltpu.sync_copy(colp_hbm.at[wid, sl], colv.at[buf])
        pltpu.sync_copy(wp_hbm.at[wid, sl], wv.at[buf])

    def zb_body(i, _):
        for v in range(D // LANES):
            rows[0, i, pl.ds(v * LANES, LANES)] = jnp.zeros((LANES,), jnp.float32)
        return 0
    lax.fori_loop(0, ZR, zb_body, 0)
    for k in range(RPT // ZR):
        pltpu.sync_copy(rows.at[0], acc.at[pl.ds(sid * RPT + k * ZR, ZR)])
    plsc.subcore_barrier()

    def gather(c, p):
        gp = (c // GRP) % 2
        pltpu.async_copy(
            xs_hbm.at[colv.at[gp, c % GRP]], rows.at[p], gsem.at[p])

    def gather_wait(c, p):
        gp = (c // GRP) % 2
        pltpu.make_async_copy(
            xs_hbm.at[colv.at[gp, c % GRP]], rows.at[p], gsem.at[p]).wait()

    def scatter(c, p):
        gp = (c // GRP) % 2
        pltpu.async_copy(
            rows.at[p], acc.at[rowv.at[gp, c % GRP]], ssem.at[p], add=True)

    def scatter_wait(c, p):
        gp = (c // GRP) % 2
        pltpu.make_async_copy(
            rows.at[p], acc.at[rowv.at[gp, c % GRP]], ssem.at[p]).wait()

    stage(0, 0)
    gather(0, 0)

    def ch_body(c, _):
        p = lax.rem(c, 2)
        gp = lax.rem(lax.div(c, GRP), 2)
        gather_wait(c, p)

        def scale_body(j, _):
            wvec = wv[gp, c % GRP, pl.ds(j * LANES, LANES)]
            for l in range(LANES):
                w = wvec[l]
                k = j * LANES + l
                for v in range(D // LANES):
                    sl = pl.ds(v * LANES, LANES)
                    rows[p, k, sl] = rows[p, k, sl] * w
            return 0
        lax.fori_loop(0, CH // LANES, scale_body, 0)

        @pl.when(jnp.logical_and(c + 1 < nch, (c + 1) % GRP == 0))
        def _():
            stage((c + 1) // GRP, ((c + 1) // GRP) % 2)

        @pl.when(c + 1 < nch)
        def _():
            @pl.when(c >= 1)
            def _():
                scatter_wait(c - 1, 1 - p)
            gather(c + 1, 1 - p)

        scatter(c, p)
        return 0
    lax.fori_loop(0, nch, ch_body, 0)
    scatter_wait(nch - 2, (nch - 2) % 2)
    scatter_wait(nch - 1, (nch - 1) % 2)
    plsc.subcore_barrier()

    for k in range(RPT // ZR):
        sl = pl.ds(sid * RPT + k * ZR, ZR)
        pltpu.sync_copy(acc.at[sl], rows.at[0])
        pltpu.sync_copy(rows.at[0], out_hbm.at[cid, sl])


def _sc_side(xs, rowp, colp, wp):
    nch = rowp.shape[1]
    fn = pl.kernel(
        functools.partial(_side_body, nch),
        out_type=jax.ShapeDtypeStruct((NC, ACC_ROWS, D), jnp.float32),
        mesh=_sc_mesh(),
        scratch_types=[
            pltpu.VMEM((2, GRP, CH), jnp.int32),
            pltpu.VMEM((2, GRP, CH), jnp.int32),
            pltpu.VMEM((2, GRP, CH), jnp.float32),
            pltpu.VMEM((2, CH, D), jnp.float32),
            pltpu.VMEM_SHARED((ACC_ROWS, D), jnp.float32),
            pltpu.SemaphoreType.DMA((2,)),
            pltpu.SemaphoreType.DMA((2,)),
        ],
    )
    return fn(xs, rowp, colp, wp)


def _dinv_block(degp_ref):
    deg = degp_ref[0] + degp_ref[1]
    return jnp.where(deg > 0, lax.rsqrt(deg), 0.0)


def _prep_body(degp_ref, x_ref, xs_ref):
    xs_ref[...] = x_ref[...] * _dinv_block(degp_ref)


def _tc_prep(degp, xp):
    bn = 1024
    return pl.pallas_call(
        _prep_body,
        grid=(DEG_PAD // bn,),
        in_specs=[
            pl.BlockSpec((NC, bn, 1), lambda i: (0, i, 0)),
            pl.BlockSpec((bn, D), lambda i: (i, 0)),
        ],
        out_specs=pl.BlockSpec((bn, D), lambda i: (i, 0)),
        out_shape=jax.ShapeDtypeStruct((DEG_PAD, D), jnp.float32),
    )(degp.reshape(NC, DEG_PAD, 1), xp)


def _final_body(degp_ref, sp_ref, x_ref, w0_ref, w1_ref, out_ref):
    side = (sp_ref[0] + sp_ref[1]) * _dinv_block(degp_ref)
    bi = x_ref[...] * side
    dn = (((1,), (1,)), ((), ()))
    out_ref[...] = (
        lax.dot_general(side, w0_ref[...], dn, preferred_element_type=jnp.float32)
        + lax.dot_general(bi, w1_ref[...], dn, preferred_element_type=jnp.float32)
    )


def _tc_final(degp, sidep, xp, W0, W1):
    bn = 1024
    return pl.pallas_call(
        _final_body,
        grid=(DEG_PAD // bn,),
        in_specs=[
            pl.BlockSpec((NC, bn, 1), lambda i: (0, i, 0)),
            pl.BlockSpec((NC, bn, D), lambda i: (0, i, 0)),
            pl.BlockSpec((bn, D), lambda i: (i, 0)),
            pl.BlockSpec((D, D), lambda i: (0, 0)),
            pl.BlockSpec((D, D), lambda i: (0, 0)),
        ],
        out_specs=pl.BlockSpec((bn, D), lambda i: (i, 0)),
        out_shape=jax.ShapeDtypeStruct((DEG_PAD, D), jnp.float32),
    )(degp.reshape(NC, DEG_PAD, 1), sidep, xp, W0, W1)


def kernel(x, edge_index, dist_weight, W0, W1):
    row = edge_index[0].astype(jnp.int32)
    col = edge_index[1].astype(jnp.int32)
    e = row.shape[0]
    ept = e // NW
    nch = -(-ept // (CH * GRP)) * GRP
    pad1 = nch * CH - ept

    row2 = row.reshape(NW, ept)
    col2 = col.reshape(NW, ept)
    w2 = dist_weight.reshape(NW, ept)
    rowp = jnp.pad(row2, ((0, 0), (0, pad1))).reshape(NW, nch, CH)
    colp_g = jnp.pad(col2, ((0, 0), (0, pad1))).reshape(NW, nch, CH)
    colp_d = jnp.pad(col2, ((0, 0), (0, pad1)),
                     constant_values=N).reshape(NW, nch, CH)
    wp = jnp.pad(w2, ((0, 0), (0, pad1))).reshape(NW, nch, CH)

    xp = jnp.pad(x, ((0, DEG_PAD - N), (0, 0)))
    degp = _sc_deg(colp_d)
    xs = _tc_prep(degp, xp)
    sidep = _sc_side(xs, rowp, colp_g, wp)
    return _tc_final(degp, sidep, xp, W0, W1)[:N]


# speedup vs baseline: 1.4601x; 1.4601x over previous
import functools

import jax
import jax.numpy as jnp
from jax import lax
from jax.experimental import pallas as pl
from jax.experimental.pallas import tpu as pltpu
from jax.experimental.pallas import tpu_sc as plsc

NC = 2
NS = 16
NW = NC * NS
CH = 128
LANES = 16

N = 10000
D = 128
DEG_PAD = 10240
DEG_SLICE = DEG_PAD // NS
ACC_ROWS = 10240
RPT = ACC_ROWS // NS
ZR = 128


def _sc_mesh():
    return plsc.VectorSubcoreMesh(core_axis_name="c", subcore_axis_name="s")


def _zero_1d(ref, n):
    def body(i, _):
        ref[pl.ds(i * LANES, LANES)] = jnp.zeros((LANES,), jnp.float32)
        return 0
    lax.fori_loop(0, n // LANES, body, 0)


def _deg_body(nch, colp_hbm, out_hbm, colv, ones_v, zb, deg_sh):
    cid = lax.axis_index("c")
    sid = lax.axis_index("s")
    wid = sid * NC + cid
    pltpu.sync_copy(colp_hbm.at[wid], colv)
    _zero_1d(zb, DEG_SLICE)
    pltpu.sync_copy(zb, deg_sh.at[pl.ds(sid * DEG_SLICE, DEG_SLICE)])
    def ones_body(i, _):
        ones_v[pl.ds(i * LANES, LANES)] = jnp.ones((LANES,), jnp.float32)
        return 0
    lax.fori_loop(0, CH // LANES, ones_body, 0)
    plsc.subcore_barrier()
    def ch_body(c, _):
        pltpu.sync_copy(ones_v, deg_sh.at[colv.at[c]], add=True)
        return 0
    lax.fori_loop(0, nch, ch_body, 0)
    plsc.subcore_barrier()
    pltpu.sync_copy(deg_sh.at[pl.ds(sid * DEG_SLICE, DEG_SLICE)], zb)
    pltpu.sync_copy(zb, out_hbm.at[cid, pl.ds(sid * DEG_SLICE, DEG_SLICE)])


def _sc_deg(colp):
    nch = colp.shape[1]
    fn = pl.kernel(
        functools.partial(_deg_body, nch),
        out_type=jax.ShapeDtypeStruct((NC, DEG_PAD), jnp.float32),
        mesh=_sc_mesh(),
        scratch_types=[
            pltpu.VMEM((nch, CH), jnp.int32),
            pltpu.VMEM((CH,), jnp.float32),
            pltpu.VMEM((DEG_SLICE,), jnp.float32),
            pltpu.VMEM_SHARED((DEG_PAD,), jnp.float32),
        ],
    )
    return fn(colp)


GRP = 16


def _side_body(nch, xs_hbm, rowp_hbm, colp_hbm, wp_hbm, out_hbm,
               rowv, colv, wv, rows, acc):
    cid = lax.axis_index("c")
    sid = lax.axis_index("s")
    wid = sid * NC + cid
    pltpu.sync_copy(rowp_hbm.at[wid], rowv)
    pltpu.sync_copy(colp_hbm.at[wid], colv)
    pltpu.sync_copy(wp_hbm.at[wid], wv)
    def zb_body(i, _):
        for v in range(D // LANES):
            rows[i, pl.ds(v * LANES, LANES)] = jnp.zeros((LANES,), jnp.float32)
        return 0
    lax.fori_loop(0, ZR, zb_body, 0)
    for k in range(RPT // ZR):
        pltpu.sync_copy(rows, acc.at[pl.ds(sid * RPT + k * ZR, ZR)])
    plsc.subcore_barrier()

    def ch_body(c, _):
        pltpu.sync_copy(xs_hbm.at[colv.at[c]], rows)
        def scale_body(j, _):
            wvec = wv[c, pl.ds(j * LANES, LANES)]
            for l in range(LANES):
                w = wvec[l]
                k = j * LANES + l
                for v in range(D // LANES):
                    sl = pl.ds(v * LANES, LANES)
                    rows[k, sl] = rows[k, sl] * w
            return 0
        lax.fori_loop(0, CH // LANES, scale_body, 0)
        pltpu.sync_copy(rows, acc.at[rowv.at[c]], add=True)
        return 0
    lax.fori_loop(0, nch, ch_body, 0)
    plsc.subcore_barrier()

    for k in range(RPT // ZR):
        sl = pl.ds(sid * RPT + k * ZR, ZR)
        pltpu.sync_copy(acc.at[sl], rows)
        pltpu.sync_copy(rows, out_hbm.at[cid, sl])


def _sc_side(xs, rowp, colp, wp):
    nch = rowp.shape[1]
    fn = pl.kernel(
        functools.partial(_side_body, nch),
        out_type=jax.ShapeDtypeStruct((NC, ACC_ROWS, D), jnp.float32),
        mesh=_sc_mesh(),
        scratch_types=[
            pltpu.VMEM((nch, CH), jnp.int32),
            pltpu.VMEM((nch, CH), jnp.int32),
            pltpu.VMEM((nch, CH), jnp.float32),
            pltpu.VMEM((CH, D), jnp.float32),
            pltpu.VMEM_SHARED((ACC_ROWS, D), jnp.float32),
        ],
    )
    return fn(xs, rowp, colp, wp)


def _dinv_block(degp_ref):
    deg = degp_ref[0] + degp_ref[1]
    return jnp.where(deg > 0, lax.rsqrt(deg), 0.0)


def _prep_body(degp_ref, x_ref, xs_ref):
    xs_ref[...] = x_ref[...] * _dinv_block(degp_ref)


def _tc_prep(degp, xp):
    bn = 1024
    return pl.pallas_call(
        _prep_body,
        grid=(DEG_PAD // bn,),
        in_specs=[
            pl.BlockSpec((NC, bn, 1), lambda i: (0, i, 0)),
            pl.BlockSpec((bn, D), lambda i: (i, 0)),
        ],
        out_specs=pl.BlockSpec((bn, D), lambda i: (i, 0)),
        out_shape=jax.ShapeDtypeStruct((DEG_PAD, D), jnp.float32),
    )(degp.reshape(NC, DEG_PAD, 1), xp)


def _final_body(degp_ref, sp_ref, x_ref, w0_ref, w1_ref, out_ref):
    side = (sp_ref[0] + sp_ref[1]) * _dinv_block(degp_ref)
    bi = x_ref[...] * side
    dn = (((1,), (1,)), ((), ()))
    out_ref[...] = (
        lax.dot_general(side, w0_ref[...], dn, preferred_element_type=jnp.float32)
        + lax.dot_general(bi, w1_ref[...], dn, preferred_element_type=jnp.float32)
    )


def _tc_final(degp, sidep, xp, W0, W1):
    bn = 1024
    return pl.pallas_call(
        _final_body,
        grid=(DEG_PAD // bn,),
        in_specs=[
            pl.BlockSpec((NC, bn, 1), lambda i: (0, i, 0)),
            pl.BlockSpec((NC, bn, D), lambda i: (0, i, 0)),
            pl.BlockSpec((bn, D), lambda i: (i, 0)),
            pl.BlockSpec((D, D), lambda i: (0, 0)),
            pl.BlockSpec((D, D), lambda i: (0, 0)),
        ],
        out_specs=pl.BlockSpec((bn, D), lambda i: (i, 0)),
        out_shape=jax.ShapeDtypeStruct((DEG_PAD, D), jnp.float32),
    )(degp.reshape(NC, DEG_PAD, 1), sidep, xp, W0, W1)


def kernel(x, edge_index, dist_weight, W0, W1):
    row = edge_index[0].astype(jnp.int32)
    col = edge_index[1].astype(jnp.int32)
    e = row.shape[0]
    ept = e // NW
    nch = -(-ept // (CH * GRP)) * GRP
    pad1 = nch * CH - ept

    row2 = row.reshape(NW, ept)
    col2 = col.reshape(NW, ept)
    w2 = dist_weight.reshape(NW, ept)
    rowp = jnp.pad(row2, ((0, 0), (0, pad1))).reshape(NW, nch, CH)
    colp_g = jnp.pad(col2, ((0, 0), (0, pad1))).reshape(NW, nch, CH)
    colp_d = jnp.pad(col2, ((0, 0), (0, pad1)),
                     constant_values=N).reshape(NW, nch, CH)
    wp = jnp.pad(w2, ((0, 0), (0, pad1))).reshape(NW, nch, CH)

    xp = jnp.pad(x, ((0, DEG_PAD - N), (0, 0)))
    degp = _sc_deg(colp_d)
    xs = _tc_prep(degp, xp)
    sidep = _sc_side(xs, rowp, colp_g, wp)
    return _tc_final(degp, sidep, xp, W0, W1)[:N]


# speedup vs baseline: 1.9953x; 1.3666x over previous
import functools

import jax
import jax.numpy as jnp
from jax import lax
from jax.experimental import pallas as pl
from jax.experimental.pallas import tpu as pltpu
from jax.experimental.pallas import tpu_sc as plsc

NC = 2
NS = 16
NW = NC * NS
CH = 128
LANES = 16

N = 10000
D = 128
DEG_PAD = 10240
DEG_SLICE = DEG_PAD // NS
ACC_ROWS = 10240
RPT = ACC_ROWS // NS
ZR = 128


def _sc_mesh():
    return plsc.VectorSubcoreMesh(core_axis_name="c", subcore_axis_name="s")


def _zero_1d(ref, n):
    def body(i, _):
        ref[pl.ds(i * LANES, LANES)] = jnp.zeros((LANES,), jnp.float32)
        return 0
    lax.fori_loop(0, n // LANES, body, 0)


def _deg_body(nch, colp_hbm, out_hbm, colv, ones_v, zb, deg_sh):
    cid = lax.axis_index("c")
    sid = lax.axis_index("s")
    wid = sid * NC + cid
    pltpu.sync_copy(colp_hbm.at[wid], colv)
    _zero_1d(zb, DEG_SLICE)
    pltpu.sync_copy(zb, deg_sh.at[pl.ds(sid * DEG_SLICE, DEG_SLICE)])
    def ones_body(i, _):
        ones_v[pl.ds(i * LANES, LANES)] = jnp.ones((LANES,), jnp.float32)
        return 0
    lax.fori_loop(0, CH // LANES, ones_body, 0)
    plsc.subcore_barrier()
    def ch_body(c, _):
        pltpu.sync_copy(ones_v, deg_sh.at[colv.at[c]], add=True)
        return 0
    lax.fori_loop(0, nch, ch_body, 0)
    plsc.subcore_barrier()
    pltpu.sync_copy(deg_sh.at[pl.ds(sid * DEG_SLICE, DEG_SLICE)], zb)
    pltpu.sync_copy(zb, out_hbm.at[cid, pl.ds(sid * DEG_SLICE, DEG_SLICE)])


def _sc_deg(colp):
    nch = colp.shape[1]
    fn = pl.kernel(
        functools.partial(_deg_body, nch),
        out_type=jax.ShapeDtypeStruct((NC, DEG_PAD), jnp.float32),
        mesh=_sc_mesh(),
        scratch_types=[
            pltpu.VMEM((nch, CH), jnp.int32),
            pltpu.VMEM((CH,), jnp.float32),
            pltpu.VMEM((DEG_SLICE,), jnp.float32),
            pltpu.VMEM_SHARED((DEG_PAD,), jnp.float32),
        ],
    )
    return fn(colp)


GRP = 16


def _side_body(nch, xs_hbm, rowp_hbm, colp_hbm, wp_hbm, out_hbm,
               rowv, colv, wv, rows, acc):
    cid = lax.axis_index("c")
    sid = lax.axis_index("s")
    wid = sid * NC + cid
    pltpu.sync_copy(rowp_hbm.at[wid], rowv)
    pltpu.sync_copy(colp_hbm.at[wid], colv)
    pltpu.sync_copy(wp_hbm.at[wid], wv)
    def zb_body(i, _):
        for v in range(D // LANES):
            rows[i, pl.ds(v * LANES, LANES)] = jnp.zeros((LANES,), jnp.float32)
        return 0
    lax.fori_loop(0, ZR, zb_body, 0)
    for k in range(RPT // ZR):
        pltpu.sync_copy(rows, acc.at[pl.ds(sid * RPT + k * ZR, ZR)])
    plsc.subcore_barrier()

    def ch_body(c, _):
        pltpu.sync_copy(xs_hbm.at[colv.at[c]], rows)
        def scale_body(j, _):
            wvec = wv[c, pl.ds(j * LANES, LANES)]
            for l in range(LANES):
                w = wvec[l]
                k = j * LANES + l
                for v in range(D // LANES):
                    sl = pl.ds(v * LANES, LANES)
                    rows[k, sl] = rows[k, sl] * w
            return 0
        lax.fori_loop(0, CH // LANES, scale_body, 0)
        pltpu.sync_copy(rows, acc.at[rowv.at[c]], add=True)
        return 0
    lax.fori_loop(0, nch, ch_body, 0)
    plsc.subcore_barrier()

    for k in range(RPT // ZR):
        sl = pl.ds(sid * RPT + k * ZR, ZR)
        pltpu.sync_copy(acc.at[sl], rows)
        pltpu.sync_copy(rows, out_hbm.at[cid, sl])


def _sc_side(xs, rowp, colp, wp):
    nch = rowp.shape[1]
    fn = pl.kernel(
        functools.partial(_side_body, nch),
        out_type=jax.ShapeDtypeStruct((NC, ACC_ROWS, D), jnp.float32),
        mesh=_sc_mesh(),
        scratch_types=[
            pltpu.VMEM((nch, CH), jnp.int32),
            pltpu.VMEM((nch, CH), jnp.int32),
            pltpu.VMEM((nch, CH), jnp.float32),
            pltpu.VMEM((CH, D), jnp.float32),
            pltpu.VMEM_SHARED((ACC_ROWS, D), jnp.float32),
        ],
    )
    return fn(xs, rowp, colp, wp)


def _dinv_block(degp_ref):
    deg = degp_ref[0] + degp_ref[1]
    return jnp.where(deg > 0, lax.rsqrt(deg), 0.0)


def _prep_body(degp_ref, x_ref, xs_ref):
    xs_ref[...] = x_ref[...] * _dinv_block(degp_ref)


def _tc_prep(degp, xp):
    bn = 1024
    return pl.pallas_call(
        _prep_body,
        grid=(DEG_PAD // bn,),
        in_specs=[
            pl.BlockSpec((NC, bn, 1), lambda i: (0, i, 0)),
            pl.BlockSpec((bn, D), lambda i: (i, 0)),
        ],
        out_specs=pl.BlockSpec((bn, D), lambda i: (i, 0)),
        out_shape=jax.ShapeDtypeStruct((DEG_PAD, D), jnp.float32),
    )(degp.reshape(NC, DEG_PAD, 1), xp)


def _final_body(degp_ref, sp_ref, x_ref, w0_ref, w1_ref, out_ref):
    side = (sp_ref[0] + sp_ref[1]) * _dinv_block(degp_ref)
    bi = x_ref[...] * side
    dn = (((1,), (1,)), ((), ()))
    out_ref[...] = (
        lax.dot_general(side, w0_ref[...], dn, preferred_element_type=jnp.float32)
        + lax.dot_general(bi, w1_ref[...], dn, preferred_element_type=jnp.float32)
    )


def _tc_final(degp, sidep, xp, W0, W1):
    bn = 1024
    return pl.pallas_call(
        _final_body,
        grid=(DEG_PAD // bn,),
        in_specs=[
            pl.BlockSpec((NC, bn, 1), lambda i: (0, i, 0)),
            pl.BlockSpec((NC, bn, D), lambda i: (0, i, 0)),
            pl.BlockSpec((bn, D), lambda i: (i, 0)),
            pl.BlockSpec((D, D), lambda i: (0, 0)),
            pl.BlockSpec((D, D), lambda i: (0, 0)),
        ],
        out_specs=pl.BlockSpec((bn, D), lambda i: (i, 0)),
        out_shape=jax.ShapeDtypeStruct((DEG_PAD, D), jnp.float32),
    )(degp.reshape(NC, DEG_PAD, 1), sidep, xp, W0, W1)


def kernel(x, edge_index, dist_weight, W0, W1):
    row = edge_index[0].astype(jnp.int32)
    col = edge_index[1].astype(jnp.int32)
    e = row.shape[0]
    ept = e // NW
    nch = -(-ept // CH)
    pad1 = nch * CH - ept

    row2 = row.reshape(NW, ept)
    col2 = col.reshape(NW, ept)
    w2 = dist_weight.reshape(NW, ept)
    rowp = jnp.pad(row2, ((0, 0), (0, pad1))).reshape(NW, nch, CH)
    colp_g = jnp.pad(col2, ((0, 0), (0, pad1))).reshape(NW, nch, CH)
    colp_d = jnp.pad(col2, ((0, 0), (0, pad1)),
                     constant_values=N).reshape(NW, nch, CH)
    wp = jnp.pad(w2, ((0, 0), (0, pad1))).reshape(NW, nch, CH)

    xp = jnp.pad(x, ((0, DEG_PAD - N), (0, 0)))
    degp = _sc_deg(colp_d)
    xs = _tc_prep(degp, xp)
    sidep = _sc_side(xs, rowp, colp_g, wp)
    return _tc_final(degp, sidep, xp, W0, W1)[:N]
